# baseline (speedup 1.0000x reference)
import jax
import jax.numpy as jnp
from jax import lax
from jax.experimental import pallas as pl

_NUM_PILOT = 512
_ROWS_PER_BLOCK = 16


def _thresh_body(w8_ref, t_ref):
    u = lax.bitcast_convert_type(w8_ref[...], jnp.uint32)
    top = jnp.uint32(0x80000000)
    key = jnp.where(u >= top, ~u, u | top)

    def body(_, carry):
        p, bit = carry
        cand = p | bit
        cnt = jnp.sum((key >= cand).astype(jnp.int32), keepdims=True).reshape(1, 1)
        return jnp.where(cnt >= _NUM_PILOT, cand, p), lax.shift_right_logical(
            bit, jnp.uint32(1)
        )

    p0 = jnp.zeros((1, 1), jnp.uint32)
    b0 = jnp.full((1, 1), top, jnp.uint32)
    p, _ = lax.fori_loop(0, 32, body, (p0, b0))
    t = jnp.where(p >= top, p ^ top, ~p)
    t_ref[...] = lax.bitcast_convert_type(t, jnp.float32)


def _mul_body(x_ref, w_ref, t_ref, o_ref):
    mask = (w_ref[...] > t_ref[...]).astype(jnp.float32)
    o_ref[...] = x_ref[...] * mask


def kernel(inputs, kernel):
    b, d = inputs.shape
    w8 = kernel.reshape(8, d // 8)
    thresh = pl.pallas_call(
        _thresh_body,
        out_shape=jax.ShapeDtypeStruct((1, 1), jnp.float32),
    )(w8)
    grid = b // _ROWS_PER_BLOCK
    out = pl.pallas_call(
        _mul_body,
        grid=(grid,),
        in_specs=[
            pl.BlockSpec((_ROWS_PER_BLOCK, d), lambda i: (i, 0)),
            pl.BlockSpec((1, d), lambda i: (0, 0)),
            pl.BlockSpec((1, 1), lambda i: (0, 0)),
        ],
        out_specs=pl.BlockSpec((_ROWS_PER_BLOCK, d), lambda i: (i, 0)),
        out_shape=jax.ShapeDtypeStruct(inputs.shape, inputs.dtype),
    )(inputs, kernel, thresh)
    return out


# speedup vs baseline: 1.8782x; 1.8782x over previous
import jax
import jax.numpy as jnp
from jax import lax
from jax.experimental import pallas as pl

_NUM_PILOT = 512


def _find_thresh(w8):
    u = lax.bitcast_convert_type(w8, jnp.uint32)
    top = jnp.uint32(0x80000000)
    key = jnp.where(u >= top, ~u, u | top)

    p = jnp.zeros((1, 1), jnp.uint32)
    bit = jnp.full((1, 1), jnp.uint32(1) << 28, jnp.uint32)
    for _ in range(8):
        m = jnp.zeros((1, 1), jnp.uint32)
        for j in range(1, 16):
            cand = p + jnp.uint32(j) * bit
            cnt = jnp.sum((key >= cand).astype(jnp.int32), keepdims=True).reshape(1, 1)
            m = m + (cnt >= _NUM_PILOT).astype(jnp.uint32)
        p = p + m * bit
        bit = lax.shift_right_logical(bit, jnp.uint32(4))
    t = jnp.where(p >= top, p ^ top, ~p)
    return lax.bitcast_convert_type(t, jnp.float32)


def _mask_mul_body(x_ref, w_ref, w8_ref, o_ref):
    thresh = _find_thresh(w8_ref[...])
    mask = (w_ref[...] > thresh).astype(jnp.float32)
    o_ref[...] = x_ref[...] * mask


def kernel(inputs, kernel):
    b, d = inputs.shape
    w8 = kernel.reshape(8, d // 8)
    out = pl.pallas_call(
        _mask_mul_body,
        out_shape=jax.ShapeDtypeStruct(inputs.shape, inputs.dtype),
    )(inputs, kernel, w8)
    return out
